# Initial kernel scaffold; baseline (speedup 1.0000x reference)
import functools

import jax
import jax.numpy as jnp
from jax import lax
from jax.experimental import pallas as pl
from jax.experimental.pallas import tpu as pltpu
from jax.experimental.pallas import tpu_sc as plsc

N_NODES = 10000
N_EDGES = 320000
D = 128

NC = 2
NS = 16
NW = NC * NS
B = 128
NCHUNK = N_EDGES // B
MAX_ITERS = (NCHUNK + NW - 1) // NW
ROWS_PER_TILE = N_NODES // NS
CW = 16

_mesh = plsc.VectorSubcoreMesh(core_axis_name="c", subcore_axis_name="s")


@functools.partial(
    pl.kernel,
    out_type=(
        jax.ShapeDtypeStruct((NC, N_NODES, D), jnp.float32),
        jax.ShapeDtypeStruct((NC, N_NODES, CW), jnp.float32),
    ),
    mesh=_mesh,
    scratch_types=[
        pltpu.VMEM((B,), jnp.int32),
        pltpu.VMEM((B,), jnp.int32),
        pltpu.VMEM((B, D), jnp.float32),
        pltpu.VMEM((B, CW), jnp.float32),
        pltpu.VMEM_SHARED((N_NODES, D), jnp.float32),
        pltpu.VMEM_SHARED((N_NODES, CW), jnp.float32),
        pltpu.SemaphoreType.DMA,
    ],
)
def _sc_aggregate(x_hbm, src_hbm, dst_hbm, z_acc, z_cnt,
                  acc_out, cnt_out,
                  src_idx, dst_idx, rows, ones_v, acc_sh, cnt_sh, sem):
    cid = lax.axis_index("c")
    sid = lax.axis_index("s")
    wid = sid * NC + cid

    r0 = sid * ROWS_PER_TILE
    pltpu.sync_copy(z_acc.at[pl.ds(r0, ROWS_PER_TILE)],
                    acc_sh.at[pl.ds(r0, ROWS_PER_TILE)])
    pltpu.sync_copy(z_cnt.at[pl.ds(r0, ROWS_PER_TILE)],
                    cnt_sh.at[pl.ds(r0, ROWS_PER_TILE)])

    def _fill(i, carry):
        ones_v[i, :] = jnp.full((CW,), 1.0, jnp.float32)
        return carry
    lax.fori_loop(0, B, _fill, 0)

    plsc.subcore_barrier()

    def _chunk(j, carry):
        c = wid + j * NW

        @pl.when(c < NCHUNK)
        def _():
            base = c * B
            pltpu.sync_copy(src_hbm.at[pl.ds(base, B)], src_idx)
            pltpu.sync_copy(dst_hbm.at[pl.ds(base, B)], dst_idx)
            pltpu.async_copy(x_hbm.at[src_idx], rows, sem).wait()
            pltpu.sync_copy(rows, acc_sh.at[dst_idx], add=True)
            pltpu.sync_copy(ones_v, cnt_sh.at[dst_idx], add=True)

        return carry

    lax.fori_loop(0, MAX_ITERS, _chunk, 0)

    plsc.subcore_barrier()

    pltpu.sync_copy(acc_sh.at[pl.ds(r0, ROWS_PER_TILE)],
                    acc_out.at[cid, pl.ds(r0, ROWS_PER_TILE)])
    pltpu.sync_copy(cnt_sh.at[pl.ds(r0, ROWS_PER_TILE)],
                    cnt_out.at[cid, pl.ds(r0, ROWS_PER_TILE)])


_TC_ROWS = 1000


def _tc_finish(acc_ref, cnt_ref, w_ref, b_ref, out_ref):
    s = acc_ref[0] + acc_ref[1]
    c = cnt_ref[0, :, 0:1] + cnt_ref[1, :, 0:1]
    m = s / jnp.maximum(c, 1.0)
    y = lax.dot_general(m, w_ref[...], (((1,), (1,)), ((), ())),
                        preferred_element_type=jnp.float32)
    out_ref[...] = jnp.where(c > 0.0, y + b_ref[...], 0.0)


def kernel(x, edge_index, W, b):
    src = edge_index[0]
    dst = edge_index[1]
    z_acc = jnp.zeros((N_NODES, D), jnp.float32)
    z_cnt = jnp.zeros((N_NODES, CW), jnp.float32)
    acc, cnt = _sc_aggregate(x, src, dst, z_acc, z_cnt)

    out = pl.pallas_call(
        _tc_finish,
        grid=(N_NODES // _TC_ROWS,),
        in_specs=[
            pl.BlockSpec((NC, _TC_ROWS, D), lambda i: (0, i, 0)),
            pl.BlockSpec((NC, _TC_ROWS, CW), lambda i: (0, i, 0)),
            pl.BlockSpec((D, D), lambda i: (0, 0)),
            pl.BlockSpec((1, D), lambda i: (0, 0)),
        ],
        out_specs=pl.BlockSpec((_TC_ROWS, D), lambda i: (i, 0)),
        out_shape=jax.ShapeDtypeStruct((N_NODES, D), jnp.float32),
    )(acc, cnt, W, b.reshape(1, D))
    return out


# speedup vs baseline: 5.9847x; 5.9847x over previous
import functools

import jax
import jax.numpy as jnp
from jax import lax
from jax.experimental import pallas as pl
from jax.experimental.pallas import tpu as pltpu
from jax.experimental.pallas import tpu_sc as plsc

N_NODES = 10000
N_EDGES = 320000
D = 128

NC = 2
NS = 16
NW = NC * NS
B = 128
NCHUNK = N_EDGES // B
NP = 10240
RPT = NP // NS
L = 16


def _sc_aggregate(x_hbm, src_hbm, dst_hbm,
                  acc_out, cnt_out,
                  src_idx, dst_idx, rows, ones_v, tab_sh, sem):
    cid = lax.axis_index("c")
    sid = lax.axis_index("s")
    wid = sid * NC + cid
    r0 = sid * RPT

    def _set_rows(ref, val):
        def _row(i, carry):
            def _col(j, carry2):
                ref[i, pl.ds(j * L, L)] = jnp.full((L,), val, jnp.float32)
                return carry2
            lax.fori_loop(0, D // L, _col, 0)
            return carry
        lax.fori_loop(0, B, _row, 0)

    def _zero_table():
        for k in range(RPT // B):
            pltpu.sync_copy(rows, tab_sh.at[pl.ds(r0 + k * B, B)])

    def _dump_table(out_ref):
        for k in range(RPT // B):
            pltpu.sync_copy(tab_sh.at[pl.ds(r0 + k * B, B)], rows)
            pltpu.sync_copy(rows, out_ref.at[cid, pl.ds(r0 + k * B, B)])

    _set_rows(rows, 0.0)
    _set_rows(ones_v, 1.0)
    _zero_table()
    plsc.subcore_barrier()

    lo = (wid * NCHUNK) // NW
    hi = ((wid + 1) * NCHUNK) // NW

    def _sum_chunk(c, carry):
        base = c * B
        pltpu.sync_copy(src_hbm.at[pl.ds(base, B)], src_idx)
        pltpu.sync_copy(dst_hbm.at[pl.ds(base, B)], dst_idx)
        pltpu.async_copy(x_hbm.at[src_idx], rows, sem).wait()
        pltpu.sync_copy(rows, tab_sh.at[dst_idx], add=True)
        return carry

    lax.fori_loop(lo, hi, _sum_chunk, 0)
    plsc.subcore_barrier()
    _dump_table(acc_out)

    _set_rows(rows, 0.0)
    _zero_table()
    plsc.subcore_barrier()

    def _cnt_chunk(c, carry):
        base = c * B
        pltpu.sync_copy(dst_hbm.at[pl.ds(base, B)], dst_idx)
        pltpu.sync_copy(ones_v, tab_sh.at[dst_idx], add=True)
        return carry

    lax.fori_loop(lo, hi, _cnt_chunk, 0)
    plsc.subcore_barrier()
    _dump_table(cnt_out)


@functools.cache
def _sc_call():
    mesh = plsc.VectorSubcoreMesh(core_axis_name="c", subcore_axis_name="s",
                                  num_cores=NC, num_subcores=NS)
    return pl.kernel(
        _sc_aggregate,
        out_type=(
            jax.ShapeDtypeStruct((NC, NP, D), jnp.float32),
            jax.ShapeDtypeStruct((NC, NP, D), jnp.float32),
        ),
        mesh=mesh,
        scratch_types=[
            pltpu.VMEM((B,), jnp.int32),
            pltpu.VMEM((B,), jnp.int32),
            pltpu.VMEM((B, D), jnp.float32),
            pltpu.VMEM((B, D), jnp.float32),
            pltpu.VMEM_SHARED((NP, D), jnp.float32),
            pltpu.SemaphoreType.DMA,
        ],
    )


_TC_ROWS = 1000


def _tc_finish(acc_ref, cnt_ref, w_ref, b_ref, out_ref):
    s = acc_ref[0] + acc_ref[1]
    c = cnt_ref[0, :, 0:1] + cnt_ref[1, :, 0:1]
    m = s / jnp.maximum(c, 1.0)
    y = lax.dot_general(m, w_ref[...], (((1,), (1,)), ((), ())),
                        preferred_element_type=jnp.float32)
    out_ref[...] = jnp.where(c > 0.0, y + b_ref[...], 0.0)


def kernel(x, edge_index, W, b):
    src = edge_index[0]
    dst = edge_index[1]
    acc, cnt = _sc_call()(x, src, dst)

    out = pl.pallas_call(
        _tc_finish,
        grid=(N_NODES // _TC_ROWS,),
        in_specs=[
            pl.BlockSpec((NC, _TC_ROWS, D), lambda i: (0, i, 0)),
            pl.BlockSpec((NC, _TC_ROWS, D), lambda i: (0, i, 0)),
            pl.BlockSpec((D, D), lambda i: (0, 0)),
            pl.BlockSpec((1, D), lambda i: (0, 0)),
        ],
        out_specs=pl.BlockSpec((_TC_ROWS, D), lambda i: (i, 0)),
        out_shape=jax.ShapeDtypeStruct((N_NODES, D), jnp.float32),
    )(acc, cnt, W, b.reshape(1, D))
    return out


# speedup vs baseline: 8.7466x; 1.4615x over previous
import functools

import jax
import jax.numpy as jnp
from jax import lax
from jax.experimental import pallas as pl
from jax.experimental.pallas import tpu as pltpu
from jax.experimental.pallas import tpu_sc as plsc

N_NODES = 10000
N_EDGES = 320000
D = 128

NC = 2
NS = 16
NW = NC * NS
B = 128
NCHUNK = N_EDGES // B
CPT = 78
TB = 16
TAIL0 = NW * CPT * B
NP = 10240
RPT = NP // NS
L = 16


def _sc_aggregate(x_hbm, src_hbm, dst_hbm,
                  acc_out, cnt_out,
                  srcA, srcB, dstA, dstB, rowsA, rowsB,
                  srcT, dstT, rowsT, tab_sh,
                  semGA, semGB, semI):
    cid = lax.axis_index("c")
    sid = lax.axis_index("s")
    wid = sid * NC + cid
    r0 = sid * RPT
    lo = wid * CPT

    def _set_rows(ref, val):
        def _row(i, carry):
            def _col(j, carry2):
                ref[i, pl.ds(j * L, L)] = jnp.full((L,), val, jnp.float32)
                return carry2
            lax.fori_loop(0, D // L, _col, 0)
            return carry
        lax.fori_loop(0, B, _row, 0)

    def _zero_table(zbuf):
        for k in range(RPT // B):
            pltpu.sync_copy(zbuf, tab_sh.at[pl.ds(r0 + k * B, B)])

    def _dump_table(out_ref, sbuf):
        for k in range(RPT // B):
            pltpu.sync_copy(tab_sh.at[pl.ds(r0 + k * B, B)], sbuf)
            pltpu.sync_copy(sbuf, out_ref.at[cid, pl.ds(r0 + k * B, B)])

    _set_rows(rowsA, 0.0)
    _zero_table(rowsA)
    plsc.subcore_barrier()

    bufs = [(srcA, dstA, rowsA, semGA), (srcB, dstB, rowsB, semGB)]

    pltpu.sync_copy(src_hbm.at[pl.ds(lo * B, B)], srcA)
    pltpu.sync_copy(dst_hbm.at[pl.ds(lo * B, B)], dstA)
    pltpu.async_copy(x_hbm.at[srcA], rowsA, semGA)
    pltpu.async_copy(src_hbm.at[pl.ds((lo + 1) * B, B)], srcB, semI)
    pltpu.async_copy(dst_hbm.at[pl.ds((lo + 1) * B, B)], dstB, semI)

    def _chunk_step(p, k, issue_gather, prefetch_idx):
        sp, dp, rp, gp = bufs[p]
        sq, dq, rq, gq = bufs[1 - p]
        if issue_gather:
            pltpu.make_async_copy(
                src_hbm.at[pl.ds((lo + k + 1) * B, B)], sq, semI).wait()
            pltpu.make_async_copy(
                dst_hbm.at[pl.ds((lo + k + 1) * B, B)], dq, semI).wait()
            pltpu.async_copy(x_hbm.at[sq], rq, gq)
        pltpu.make_async_copy(x_hbm.at[sp], rp, gp).wait()
        pltpu.sync_copy(rp, tab_sh.at[dp], add=True)
        if prefetch_idx:
            pltpu.async_copy(src_hbm.at[pl.ds((lo + k + 2) * B, B)], sp, semI)
            pltpu.async_copy(dst_hbm.at[pl.ds((lo + k + 2) * B, B)], dp, semI)

    def _pair(t, carry):
        k = t * 2
        _chunk_step(0, k, True, True)
        _chunk_step(1, k + 1, True, True)
        return carry

    lax.fori_loop(0, CPT // 2 - 1, _pair, 0)
    _chunk_step(0, CPT - 2, True, False)
    _chunk_step(1, CPT - 1, False, False)

    tb = TAIL0 + wid * TB
    pltpu.sync_copy(src_hbm.at[pl.ds(tb, TB)], srcT)
    pltpu.sync_copy(dst_hbm.at[pl.ds(tb, TB)], dstT)
    pltpu.async_copy(x_hbm.at[srcT], rowsT, semGA).wait()
    pltpu.sync_copy(rowsT, tab_sh.at[dstT], add=True)

    plsc.subcore_barrier()
    _dump_table(acc_out, rowsA)

    _set_rows(rowsA, 0.0)
    _set_rows(rowsB, 1.0)
    _zero_table(rowsA)
    plsc.subcore_barrier()

    pltpu.sync_copy(dst_hbm.at[pl.ds(lo * B, B)], dstA)
    pltpu.async_copy(dst_hbm.at[pl.ds((lo + 1) * B, B)], dstB, semI)

    def _cnt_step(p, k, wait_idx, prefetch_idx):
        dp = bufs[p][1]
        dq = bufs[1 - p][1]
        if wait_idx:
            pltpu.make_async_copy(
                dst_hbm.at[pl.ds((lo + k + 1) * B, B)], dq, semI).wait()
        pltpu.sync_copy(rowsB, tab_sh.at[dp], add=True)
        if prefetch_idx:
            pltpu.async_copy(dst_hbm.at[pl.ds((lo + k + 2) * B, B)], dp, semI)

    def _cnt_pair(t, carry):
        k = t * 2
        _cnt_step(0, k, True, True)
        _cnt_step(1, k + 1, True, True)
        return carry

    lax.fori_loop(0, CPT // 2 - 1, _cnt_pair, 0)
    _cnt_step(0, CPT - 2, True, False)
    _cnt_step(1, CPT - 1, False, False)

    pltpu.sync_copy(dst_hbm.at[pl.ds(tb, TB)], dstT)
    ones_t = rowsT
    def _fill_t(i, carry):
        def _col(j, carry2):
            ones_t[i, pl.ds(j * L, L)] = jnp.full((L,), 1.0, jnp.float32)
            return carry2
        lax.fori_loop(0, D // L, _col, 0)
        return carry
    lax.fori_loop(0, TB, _fill_t, 0)
    pltpu.sync_copy(ones_t, tab_sh.at[dstT], add=True)

    plsc.subcore_barrier()
    _dump_table(cnt_out, rowsA)


@functools.cache
def _sc_call():
    mesh = plsc.VectorSubcoreMesh(core_axis_name="c", subcore_axis_name="s",
                                  num_cores=NC, num_subcores=NS)
    return pl.kernel(
        _sc_aggregate,
        out_type=(
            jax.ShapeDtypeStruct((NC, NP, D), jnp.float32),
            jax.ShapeDtypeStruct((NC, NP, D), jnp.float32),
        ),
        mesh=mesh,
        scratch_types=[
            pltpu.VMEM((B,), jnp.int32),
            pltpu.VMEM((B,), jnp.int32),
            pltpu.VMEM((B,), jnp.int32),
            pltpu.VMEM((B,), jnp.int32),
            pltpu.VMEM((B, D), jnp.float32),
            pltpu.VMEM((B, D), jnp.float32),
            pltpu.VMEM((TB,), jnp.int32),
            pltpu.VMEM((TB,), jnp.int32),
            pltpu.VMEM((TB, D), jnp.float32),
            pltpu.VMEM_SHARED((NP, D), jnp.float32),
            pltpu.SemaphoreType.DMA,
            pltpu.SemaphoreType.DMA,
            pltpu.SemaphoreType.DMA,
        ],
    )


_TC_ROWS = 1000


def _tc_finish(acc_ref, cnt_ref, w_ref, b_ref, out_ref):
    s = acc_ref[0] + acc_ref[1]
    c = cnt_ref[0, :, 0:1] + cnt_ref[1, :, 0:1]
    m = s / jnp.maximum(c, 1.0)
    y = lax.dot_general(m, w_ref[...], (((1,), (1,)), ((), ())),
                        preferred_element_type=jnp.float32)
    out_ref[...] = jnp.where(c > 0.0, y + b_ref[...], 0.0)


def kernel(x, edge_index, W, b):
    src = edge_index[0]
    dst = edge_index[1]
    acc, cnt = _sc_call()(x, src, dst)

    out = pl.pallas_call(
        _tc_finish,
        grid=(N_NODES // _TC_ROWS,),
        in_specs=[
            pl.BlockSpec((NC, _TC_ROWS, D), lambda i: (0, i, 0)),
            pl.BlockSpec((NC, _TC_ROWS, D), lambda i: (0, i, 0)),
            pl.BlockSpec((D, D), lambda i: (0, 0)),
            pl.BlockSpec((1, D), lambda i: (0, 0)),
        ],
        out_specs=pl.BlockSpec((_TC_ROWS, D), lambda i: (i, 0)),
        out_shape=jax.ShapeDtypeStruct((N_NODES, D), jnp.float32),
    )(acc, cnt, W, b.reshape(1, D))
    return out
